# Initial kernel scaffold; baseline (speedup 1.0000x reference)
import jax
import jax.numpy as jnp
from jax.experimental import pallas as pl


def kernel(x, edge_index, pre_W, pre_b, pre_g, pre_be, c0_W, c0_b, bn0_g, bn0_be, c1_W, c1_b, bn1_g, bn1_be, c2_W, c2_b, bn2_g, bn2_be, post_W, post_b, post_g, post_be, cls_W, cls_b):
    raise NotImplementedError("write your pallas kernel here")


# speedup vs baseline: 2.2902x; 2.2902x over previous
import functools

import jax
import jax.numpy as jnp
from jax import lax
from jax.experimental import pallas as pl
from jax.experimental.pallas import tpu as pltpu
from jax.experimental.pallas import tpu_sc as plsc

N = 10000
EDGES = 320000
LANE = 128
ROW_BLOCK = 2000
NB = N // ROW_BLOCK
N_TILES = 16
EB_ROWS = 160
EG_ROWS = 32
EG = EB_ROWS // EG_ROWS
E_PAD = N_TILES * EB_ROWS * LANE
N_PAD = 10112
ZROWS = N_PAD // N_TILES
EPS = 1e-5


def _mm_stats(x_chunks, agg_chunks, W, b, relu_first):
    n_in = len(x_chunks)
    dout = W.shape[1]
    has_agg = agg_chunks is not None

    def body(*refs):
        i = pl.program_id(0)
        x_refs = refs[:n_in]
        off = n_in
        a_refs = refs[off:off + n_in] if has_agg else ()
        off += n_in if has_agg else 0
        w_ref, b_ref = refs[off], refs[off + 1]
        u_ref, st_ref = refs[off + 2], refs[off + 3]
        acc = jnp.zeros((ROW_BLOCK, dout), jnp.float32)
        for c in range(n_in):
            xc = x_refs[c][...]
            if has_agg:
                xc = xc + a_refs[c][...]
            acc = acc + jnp.dot(xc, w_ref[c * LANE:(c + 1) * LANE, :],
                                preferred_element_type=jnp.float32)
        u = acc + b_ref[0]
        if relu_first:
            u = jnp.maximum(u, 0.0)
        u_ref[...] = u
        s0 = jnp.sum(u, axis=0)
        s1 = jnp.sum(u * u, axis=0)
        st = jnp.concatenate(
            [s0[None], s1[None], jnp.zeros((6, dout), jnp.float32)], axis=0)

        @pl.when(i == 0)
        def _():
            st_ref[...] = st

        @pl.when(i != 0)
        def _():
            st_ref[...] = st_ref[...] + st

    in_specs = [pl.BlockSpec((ROW_BLOCK, LANE), lambda i: (i, 0))
                for _ in range(n_in)]
    if has_agg:
        in_specs += [pl.BlockSpec((ROW_BLOCK, LANE), lambda i: (i, 0))
                     for _ in range(n_in)]
    in_specs += [pl.BlockSpec(W.shape, lambda i: (0, 0)),
                 pl.BlockSpec((1, dout), lambda i: (0, 0))]
    out_shape = [jax.ShapeDtypeStruct((N, dout), jnp.float32),
                 jax.ShapeDtypeStruct((8, dout), jnp.float32)]
    out_specs = [pl.BlockSpec((ROW_BLOCK, dout), lambda i: (i, 0)),
                 pl.BlockSpec((8, dout), lambda i: (0, 0))]
    args = list(x_chunks) + (list(agg_chunks) if has_agg else [])
    args += [W, b.reshape(1, dout)]
    return pl.pallas_call(
        body, grid=(NB,), in_specs=in_specs, out_specs=out_specs,
        out_shape=out_shape,
        compiler_params=pltpu.CompilerParams(
            dimension_semantics=("arbitrary",)),
    )(*args)


def _affine(u, stats, g, be, relu_after, n_out):
    dout = u.shape[1]

    def body(u_ref, st_ref, g_ref, be_ref, *out_refs):
        mean = st_ref[0] / N
        var = st_ref[1] / N - mean * mean
        scale = g_ref[0] * lax.rsqrt(var + EPS)
        shift = be_ref[0] - mean * scale
        h = u_ref[...] * scale + shift
        if relu_after:
            h = jnp.maximum(h, 0.0)
        for c in range(n_out):
            out_refs[c][...] = h[:, c * LANE:(c + 1) * LANE]

    in_specs = [pl.BlockSpec((ROW_BLOCK, dout), lambda i: (i, 0)),
                pl.BlockSpec((8, dout), lambda i: (0, 0)),
                pl.BlockSpec((1, dout), lambda i: (0, 0)),
                pl.BlockSpec((1, dout), lambda i: (0, 0))]
    out_shape = [jax.ShapeDtypeStruct((N, LANE), jnp.float32)
                 for _ in range(n_out)]
    out_specs = [pl.BlockSpec((ROW_BLOCK, LANE), lambda i: (i, 0))
                 for _ in range(n_out)]
    outs = pl.pallas_call(
        body, grid=(NB,), in_specs=in_specs, out_specs=out_specs,
        out_shape=out_shape,
        compiler_params=pltpu.CompilerParams(
            dimension_semantics=("arbitrary",)),
    )(u, stats, g.reshape(1, dout), be.reshape(1, dout))
    return list(outs)


def _mm_plain(x_chunks, Wp, bp):
    n_in = len(x_chunks)

    def body(*refs):
        x_refs = refs[:n_in]
        w_ref, b_ref, o_ref = refs[n_in], refs[n_in + 1], refs[n_in + 2]
        acc = jnp.zeros((ROW_BLOCK, LANE), jnp.float32)
        for c in range(n_in):
            acc = acc + jnp.dot(x_refs[c][...],
                                w_ref[c * LANE:(c + 1) * LANE, :],
                                preferred_element_type=jnp.float32)
        o_ref[...] = acc + b_ref[0]

    in_specs = [pl.BlockSpec((ROW_BLOCK, LANE), lambda i: (i, 0))
                for _ in range(n_in)]
    in_specs += [pl.BlockSpec(Wp.shape, lambda i: (0, 0)),
                 pl.BlockSpec((1, LANE), lambda i: (0, 0))]
    return pl.pallas_call(
        body, grid=(NB,), in_specs=in_specs,
        out_specs=pl.BlockSpec((ROW_BLOCK, LANE), lambda i: (i, 0)),
        out_shape=jax.ShapeDtypeStruct((N, LANE), jnp.float32),
    )(*x_chunks, Wp, bp.reshape(1, LANE))


def _make_seg(n_chunks):
    n_rounds = (n_chunks + 1) // 2
    out_type = [jax.ShapeDtypeStruct((N_PAD, LANE), jnp.float32)
                for _ in range(n_chunks)]
    scratch = [
        pltpu.VMEM_SHARED((N_PAD, LANE), jnp.float32),
        pltpu.VMEM((EG_ROWS, LANE), jnp.int32),
        pltpu.VMEM((EG_ROWS, LANE), jnp.int32),
        pltpu.VMEM((LANE, LANE), jnp.float32),
        pltpu.SemaphoreType.DMA,
    ]
    mesh = plsc.VectorSubcoreMesh(core_axis_name="c", subcore_axis_name="s")

    @functools.partial(pl.kernel, mesh=mesh, out_type=out_type,
                       scratch_types=scratch)
    def seg(*refs):
        h_refs = refs[:n_chunks]
        src_ref = refs[n_chunks]
        dst_ref = refs[n_chunks + 1]
        z_ref = refs[n_chunks + 2]
        out_refs = refs[n_chunks + 3:2 * n_chunks + 3]
        acc, sidx, didx, gbuf, sem = refs[2 * n_chunks + 3:]
        cid = lax.axis_index("c")
        sid = lax.axis_index("s")
        zero_base = sid * ZROWS

        def run_chunk(chunk):
            def group(g, gcarry):
                pltpu.sync_copy(src_ref.at[sid, pl.ds(g * EG_ROWS, EG_ROWS)],
                                sidx)
                pltpu.sync_copy(dst_ref.at[sid, pl.ds(g * EG_ROWS, EG_ROWS)],
                                didx)

                def step(j, carry):
                    pltpu.async_copy(h_refs[chunk].at[sidx.at[j]], gbuf,
                                     sem).wait()
                    pltpu.sync_copy(gbuf, acc.at[didx.at[j]], add=True)
                    return carry
                lax.fori_loop(0, EG_ROWS, step, 0)
                return gcarry
            lax.fori_loop(0, EG, group, 0)

        def drain_chunk(chunk):
            pltpu.sync_copy(acc.at[pl.ds(zero_base, ZROWS)],
                            out_refs[chunk].at[pl.ds(zero_base, ZROWS)])

        for r in range(n_rounds):
            pltpu.sync_copy(z_ref, gbuf)
            for k in range(4):
                pltpu.sync_copy(gbuf, acc.at[pl.ds(zero_base + k * LANE,
                                                   LANE)])
            pltpu.sync_copy(gbuf.at[pl.ds(0, ZROWS - 4 * LANE)],
                            acc.at[pl.ds(zero_base + 4 * LANE,
                                         ZROWS - 4 * LANE)])
            plsc.subcore_barrier()
            for core in range(2):
                chunk = 2 * r + core
                if chunk < n_chunks:
                    pl.when(cid == core)(functools.partial(run_chunk, chunk))
            plsc.subcore_barrier()
            for core in range(2):
                chunk = 2 * r + core
                if chunk < n_chunks:
                    pl.when(cid == core)(functools.partial(drain_chunk,
                                                           chunk))
            if r != n_rounds - 1:
                plsc.subcore_barrier()

    return seg


def _seg_sum(h_chunks, src_p, dst_p, zeros_tile):
    outs = _make_seg(len(h_chunks))(*h_chunks, src_p, dst_p, zeros_tile)
    if not isinstance(outs, (list, tuple)):
        outs = [outs]
    return list(outs)


def kernel(x, edge_index, pre_W, pre_b, pre_g, pre_be, c0_W, c0_b, bn0_g,
           bn0_be, c1_W, c1_b, bn1_g, bn1_be, c2_W, c2_b, bn2_g, bn2_be,
           post_W, post_b, post_g, post_be, cls_W, cls_b):
    src = edge_index[0].astype(jnp.int32)
    dst = edge_index[1].astype(jnp.int32)
    pad = E_PAD - EDGES
    src_p = jnp.concatenate(
        [src, jnp.zeros((pad,), jnp.int32)]).reshape(N_TILES, EB_ROWS, LANE)
    dst_p = jnp.concatenate(
        [dst, jnp.full((pad,), N, jnp.int32)]).reshape(N_TILES, EB_ROWS, LANE)
    zeros_tile = jnp.zeros((LANE, LANE), jnp.float32)

    u, st = _mm_stats([x], None, pre_W, pre_b, relu_first=True)
    h = _affine(u, st, pre_g, pre_be, relu_after=False, n_out=2)

    for W, b, g, be in ((c0_W, c0_b, bn0_g, bn0_be),
                        (c1_W, c1_b, bn1_g, bn1_be),
                        (c2_W, c2_b, bn2_g, bn2_be)):
        agg = _seg_sum(h, src_p, dst_p, zeros_tile)
        u, st = _mm_stats(h, agg, W, b, relu_first=False)
        h = _affine(u, st, g, be, relu_after=True, n_out=W.shape[1] // LANE)

    u, st = _mm_stats(h, None, post_W, post_b, relu_first=True)
    h = _affine(u, st, post_g, post_be, relu_after=False, n_out=2)

    Wp = jnp.pad(cls_W, ((0, 0), (0, LANE - cls_W.shape[1])))
    bp = jnp.pad(cls_b, (0, LANE - cls_b.shape[0]))
    out = _mm_plain(h, Wp, bp)
    return out[:, :cls_W.shape[1]]


# speedup vs baseline: 7.0341x; 3.0714x over previous
import functools

import jax
import jax.numpy as jnp
from jax import lax
from jax.experimental import pallas as pl
from jax.experimental.pallas import tpu as pltpu
from jax.experimental.pallas import tpu_sc as plsc

N = 10000
EDGES = 320000
LANE = 128
ROW_BLOCK = 2000
NB = N // ROW_BLOCK
N_TILES = 16
EB_ROWS = 160
EG_ROWS = 32
EG = EB_ROWS // EG_ROWS
E_PAD = N_TILES * EB_ROWS * LANE
N_PAD = 10112
ZROWS = N_PAD // N_TILES
EPS = 1e-5


def _mm_stats(x_chunks, agg_chunks, W, b, relu_first):
    n_in = len(x_chunks)
    dout = W.shape[1]
    has_agg = agg_chunks is not None

    def body(*refs):
        i = pl.program_id(0)
        x_refs = refs[:n_in]
        off = n_in
        a_refs = refs[off:off + n_in] if has_agg else ()
        off += n_in if has_agg else 0
        w_ref, b_ref = refs[off], refs[off + 1]
        u_ref, st_ref = refs[off + 2], refs[off + 3]
        acc = jnp.zeros((ROW_BLOCK, dout), jnp.float32)
        for c in range(n_in):
            xc = x_refs[c][...]
            if has_agg:
                xc = xc + a_refs[c][...]
            acc = acc + jnp.dot(xc, w_ref[c * LANE:(c + 1) * LANE, :],
                                preferred_element_type=jnp.float32)
        u = acc + b_ref[0]
        if relu_first:
            u = jnp.maximum(u, 0.0)
        u_ref[...] = u
        s0 = jnp.sum(u, axis=0)
        s1 = jnp.sum(u * u, axis=0)
        st = jnp.concatenate(
            [s0[None], s1[None], jnp.zeros((6, dout), jnp.float32)], axis=0)

        @pl.when(i == 0)
        def _():
            st_ref[...] = st

        @pl.when(i != 0)
        def _():
            st_ref[...] = st_ref[...] + st

    in_specs = [pl.BlockSpec((ROW_BLOCK, LANE), lambda i: (i, 0))
                for _ in range(n_in)]
    if has_agg:
        in_specs += [pl.BlockSpec((ROW_BLOCK, LANE), lambda i: (i, 0))
                     for _ in range(n_in)]
    in_specs += [pl.BlockSpec(W.shape, lambda i: (0, 0)),
                 pl.BlockSpec((1, dout), lambda i: (0, 0))]
    out_shape = [jax.ShapeDtypeStruct((N, dout), jnp.float32),
                 jax.ShapeDtypeStruct((8, dout), jnp.float32)]
    out_specs = [pl.BlockSpec((ROW_BLOCK, dout), lambda i: (i, 0)),
                 pl.BlockSpec((8, dout), lambda i: (0, 0))]
    args = list(x_chunks) + (list(agg_chunks) if has_agg else [])
    args += [W, b.reshape(1, dout)]
    return pl.pallas_call(
        body, grid=(NB,), in_specs=in_specs, out_specs=out_specs,
        out_shape=out_shape,
        compiler_params=pltpu.CompilerParams(
            dimension_semantics=("arbitrary",)),
    )(*args)


def _affine(u, stats, g, be, relu_after, n_out):
    dout = u.shape[1]

    def body(u_ref, st_ref, g_ref, be_ref, *out_refs):
        mean = st_ref[0] / N
        var = st_ref[1] / N - mean * mean
        scale = g_ref[0] * lax.rsqrt(var + EPS)
        shift = be_ref[0] - mean * scale
        h = u_ref[...] * scale + shift
        if relu_after:
            h = jnp.maximum(h, 0.0)
        for c in range(n_out):
            out_refs[c][...] = h[:, c * LANE:(c + 1) * LANE]

    in_specs = [pl.BlockSpec((ROW_BLOCK, dout), lambda i: (i, 0)),
                pl.BlockSpec((8, dout), lambda i: (0, 0)),
                pl.BlockSpec((1, dout), lambda i: (0, 0)),
                pl.BlockSpec((1, dout), lambda i: (0, 0))]
    out_shape = [jax.ShapeDtypeStruct((N, LANE), jnp.float32)
                 for _ in range(n_out)]
    out_specs = [pl.BlockSpec((ROW_BLOCK, LANE), lambda i: (i, 0))
                 for _ in range(n_out)]
    outs = pl.pallas_call(
        body, grid=(NB,), in_specs=in_specs, out_specs=out_specs,
        out_shape=out_shape,
        compiler_params=pltpu.CompilerParams(
            dimension_semantics=("arbitrary",)),
    )(u, stats, g.reshape(1, dout), be.reshape(1, dout))
    return list(outs)


def _mm_plain(x_chunks, Wp, bp):
    n_in = len(x_chunks)

    def body(*refs):
        x_refs = refs[:n_in]
        w_ref, b_ref, o_ref = refs[n_in], refs[n_in + 1], refs[n_in + 2]
        acc = jnp.zeros((ROW_BLOCK, LANE), jnp.float32)
        for c in range(n_in):
            acc = acc + jnp.dot(x_refs[c][...],
                                w_ref[c * LANE:(c + 1) * LANE, :],
                                preferred_element_type=jnp.float32)
        o_ref[...] = acc + b_ref[0]

    in_specs = [pl.BlockSpec((ROW_BLOCK, LANE), lambda i: (i, 0))
                for _ in range(n_in)]
    in_specs += [pl.BlockSpec(Wp.shape, lambda i: (0, 0)),
                 pl.BlockSpec((1, LANE), lambda i: (0, 0))]
    return pl.pallas_call(
        body, grid=(NB,), in_specs=in_specs,
        out_specs=pl.BlockSpec((ROW_BLOCK, LANE), lambda i: (i, 0)),
        out_shape=jax.ShapeDtypeStruct((N, LANE), jnp.float32),
    )(*x_chunks, Wp, bp.reshape(1, LANE))


def _make_seg(n_chunks):
    n_rounds = (n_chunks + 1) // 2
    out_type = [jax.ShapeDtypeStruct((N_PAD, LANE), jnp.float32)
                for _ in range(n_chunks)]
    scratch = [
        pltpu.VMEM_SHARED((N_PAD, LANE), jnp.float32),
        pltpu.VMEM((EG_ROWS, LANE), jnp.int32),
        pltpu.VMEM((EG_ROWS, LANE), jnp.int32),
        pltpu.VMEM((LANE, LANE), jnp.float32),
        pltpu.VMEM((LANE, LANE), jnp.float32),
        pltpu.SemaphoreType.DMA,
        pltpu.SemaphoreType.DMA,
    ]
    mesh = plsc.VectorSubcoreMesh(core_axis_name="c", subcore_axis_name="s")

    @functools.partial(pl.kernel, mesh=mesh, out_type=out_type,
                       scratch_types=scratch)
    def seg(*refs):
        h_refs = refs[:n_chunks]
        src_ref = refs[n_chunks]
        dst_ref = refs[n_chunks + 1]
        z_ref = refs[n_chunks + 2]
        out_refs = refs[n_chunks + 3:2 * n_chunks + 3]
        acc, sidx, didx, gbuf, gbuf2, sem, sem2 = refs[2 * n_chunks + 3:]
        cid = lax.axis_index("c")
        sid = lax.axis_index("s")
        zero_base = sid * ZROWS

        def run_chunk(chunk):
            h_ref = h_refs[chunk]

            def group(g, gcarry):
                pltpu.sync_copy(src_ref.at[sid, pl.ds(g * EG_ROWS, EG_ROWS)],
                                sidx)
                pltpu.sync_copy(dst_ref.at[sid, pl.ds(g * EG_ROWS, EG_ROWS)],
                                didx)
                pltpu.make_async_copy(h_ref.at[sidx.at[0]], gbuf, sem).start()

                def pair(t, carry):
                    j0 = 2 * t
                    pltpu.make_async_copy(h_ref.at[sidx.at[j0 + 1]], gbuf2,
                                          sem2).start()
                    pltpu.make_async_copy(h_ref.at[sidx.at[j0]], gbuf,
                                          sem).wait()
                    pltpu.sync_copy(gbuf, acc.at[didx.at[j0]], add=True)

                    @pl.when(t + 1 < EG_ROWS // 2)
                    def _():
                        pltpu.make_async_copy(h_ref.at[sidx.at[j0 + 2]],
                                              gbuf, sem).start()
                    pltpu.make_async_copy(h_ref.at[sidx.at[j0 + 1]], gbuf2,
                                          sem2).wait()
                    pltpu.sync_copy(gbuf2, acc.at[didx.at[j0 + 1]], add=True)
                    return carry
                lax.fori_loop(0, EG_ROWS // 2, pair, 0)
                return gcarry
            lax.fori_loop(0, EG, group, 0)

        def drain_chunk(chunk):
            pltpu.sync_copy(acc.at[pl.ds(zero_base, ZROWS)],
                            out_refs[chunk].at[pl.ds(zero_base, ZROWS)])

        for r in range(n_rounds):
            pltpu.sync_copy(z_ref, gbuf)
            for k in range(4):
                pltpu.sync_copy(gbuf, acc.at[pl.ds(zero_base + k * LANE,
                                                   LANE)])
            pltpu.sync_copy(gbuf.at[pl.ds(0, ZROWS - 4 * LANE)],
                            acc.at[pl.ds(zero_base + 4 * LANE,
                                         ZROWS - 4 * LANE)])
            plsc.subcore_barrier()
            for core in range(2):
                chunk = 2 * r + core
                if chunk < n_chunks:
                    pl.when(cid == core)(functools.partial(run_chunk, chunk))
            plsc.subcore_barrier()
            for core in range(2):
                chunk = 2 * r + core
                if chunk < n_chunks:
                    pl.when(cid == core)(functools.partial(drain_chunk,
                                                           chunk))
            if r != n_rounds - 1:
                plsc.subcore_barrier()

    return seg


def _seg_sum(h_chunks, src_p, dst_p, zeros_tile):
    outs = _make_seg(len(h_chunks))(*h_chunks, src_p, dst_p, zeros_tile)
    if not isinstance(outs, (list, tuple)):
        outs = [outs]
    return list(outs)


def kernel(x, edge_index, pre_W, pre_b, pre_g, pre_be, c0_W, c0_b, bn0_g,
           bn0_be, c1_W, c1_b, bn1_g, bn1_be, c2_W, c2_b, bn2_g, bn2_be,
           post_W, post_b, post_g, post_be, cls_W, cls_b):
    src = edge_index[0].astype(jnp.int32)
    dst = edge_index[1].astype(jnp.int32)
    pad = E_PAD - EDGES
    pad_i = jnp.arange(pad, dtype=jnp.int32)
    src_p = jnp.concatenate(
        [src, pad_i % N]).reshape(N_TILES, EB_ROWS, LANE)
    dst_p = jnp.concatenate(
        [dst, N + pad_i % (N_PAD - N)]).reshape(N_TILES, EB_ROWS, LANE)
    zeros_tile = jnp.zeros((LANE, LANE), jnp.float32)

    u, st = _mm_stats([x], None, pre_W, pre_b, relu_first=True)
    h = _affine(u, st, pre_g, pre_be, relu_after=False, n_out=2)

    for W, b, g, be in ((c0_W, c0_b, bn0_g, bn0_be),
                        (c1_W, c1_b, bn1_g, bn1_be),
                        (c2_W, c2_b, bn2_g, bn2_be)):
        agg = _seg_sum(h, src_p, dst_p, zeros_tile)
        u, st = _mm_stats(h, agg, W, b, relu_first=False)
        h = _affine(u, st, g, be, relu_after=True, n_out=W.shape[1] // LANE)

    u, st = _mm_stats(h, None, post_W, post_b, relu_first=True)
    h = _affine(u, st, post_g, post_be, relu_after=False, n_out=2)

    Wp = jnp.pad(cls_W, ((0, 0), (0, LANE - cls_W.shape[1])))
    bp = jnp.pad(cls_b, (0, LANE - cls_b.shape[0]))
    out = _mm_plain(h, Wp, bp)
    return out[:, :cls_W.shape[1]]
